# Initial kernel scaffold; baseline (speedup 1.0000x reference)
import jax
import jax.numpy as jnp
from jax.experimental import pallas as pl


def kernel(x, qkv_w, qkv_b, wo_w, wo_b, lepe_w, lepe_b):
    raise NotImplementedError("write your pallas kernel here")


# speedup vs baseline: 1.9062x; 1.9062x over previous
import jax
import jax.numpy as jnp
from jax.experimental import pallas as pl
from jax.experimental.pallas import tpu as pltpu

N_WIN = 7
NUM_HEADS = 8
TOPK = 4
DIM = 192
HD = DIM // NUM_HEADS
WS = 16
W2 = WS * WS
P2 = N_WIN * N_WIN
SCALE = DIM ** -0.5
ROWS = 16
IMG = N_WIN * WS


def _qkv_kernel(x_ref, wq_ref, wkv_ref, bq_ref, bkv_ref,
                q_ref, kv_ref, vimg_ref, qwin_ref, kwin_ref):
    xw = x_ref[...].reshape(W2, DIM)
    q = jnp.dot(xw, wq_ref[...], preferred_element_type=jnp.float32) + bq_ref[...]
    kv = jnp.dot(xw, wkv_ref[...], preferred_element_type=jnp.float32) + bkv_ref[...]
    q_ref[0] = q
    kv_ref[0] = kv
    vimg_ref[...] = kv[:, DIM:].reshape(WS, WS, DIM)
    qwin_ref[0] = jnp.mean(q, axis=0, keepdims=True)
    kwin_ref[0] = jnp.mean(kv[:, :DIM], axis=0, keepdims=True)


def _router_kernel(qw_ref, kw_ref, o0, o1, o2, o3):
    qw = qw_ref[...].reshape(P2, DIM) * SCALE
    kw = kw_ref[...].reshape(P2, DIM)
    logits = jax.lax.dot_general(qw, kw, (((1,), (1,)), ((), ())),
                                 preferred_element_type=jnp.float32)
    cols = jax.lax.broadcasted_iota(jnp.int32, (P2, P2), 1)
    outs = (o0, o1, o2, o3)
    for t in range(TOPK):
        m = jnp.max(logits, axis=1, keepdims=True)
        idx = jnp.min(jnp.where(logits == m, cols, P2), axis=1, keepdims=True)
        outs[t][...] = idx
        logits = jnp.where(cols == idx, -jnp.inf, logits)


def _attn_kernel(idx_ref, q_ref, kv0, kv1, kv2, kv3, o_ref):
    kvs = (kv0, kv1, kv2, kv3)
    q = q_ref[0] * SCALE
    outs = []
    for h in range(NUM_HEADS):
        lo, hi = h * HD, (h + 1) * HD
        qh = q[:, lo:hi]
        lg = jnp.concatenate([
            jax.lax.dot_general(qh, kvs[t][0][:, lo:hi],
                                (((1,), (1,)), ((), ())),
                                preferred_element_type=jnp.float32)
            for t in range(TOPK)], axis=1)
        m = jnp.max(lg, axis=1, keepdims=True)
        p = jnp.exp(lg - m)
        p = p / jnp.sum(p, axis=1, keepdims=True)
        oh = None
        for t in range(TOPK):
            c = jax.lax.dot_general(p[:, t * W2:(t + 1) * W2],
                                    kvs[t][0][:, DIM + lo:DIM + hi],
                                    (((1,), (0,)), ((), ())),
                                    preferred_element_type=jnp.float32)
            oh = c if oh is None else oh + c
        outs.append(oh)
    o_ref[...] = jnp.concatenate(outs, axis=1).reshape(WS, WS, DIM)


def _out_kernel(attn_ref, vpad_ref, lw_ref, lb_ref, wo_ref, wob_ref, o_ref):
    i = pl.program_id(0)
    acc = attn_ref[...]
    for di in range(5):
        for dj in range(5):
            w = lw_ref[di * 5 + dj:di * 5 + dj + 1, :].reshape(1, 1, DIM)
            acc = acc + vpad_ref[pl.ds(i * ROWS + di, ROWS),
                                 pl.ds(dj, IMG), :] * w
    acc = acc + lb_ref[...].reshape(1, 1, DIM)
    y = jnp.dot(acc.reshape(ROWS * IMG, DIM), wo_ref[...],
                preferred_element_type=jnp.float32) + wob_ref[...]
    o_ref[...] = y.reshape(ROWS, IMG, DIM)


def kernel(x, qkv_w, qkv_b, wo_w, wo_b, lepe_w, lepe_b):
    B, H, W, C = x.shape
    f32 = jnp.float32
    x2 = x[0]
    wq = qkv_w[:, :DIM]
    wkv = qkv_w[:, DIM:]
    bq = qkv_b[:DIM].reshape(1, DIM)
    bkv = qkv_b[DIM:].reshape(1, 2 * DIM)

    q, kv, vimg, qwin, kwin = pl.pallas_call(
        _qkv_kernel,
        grid=(N_WIN, N_WIN),
        in_specs=[
            pl.BlockSpec((WS, WS, DIM), lambda i, j: (i, j, 0)),
            pl.BlockSpec((DIM, DIM), lambda i, j: (0, 0)),
            pl.BlockSpec((DIM, 2 * DIM), lambda i, j: (0, 0)),
            pl.BlockSpec((1, DIM), lambda i, j: (0, 0)),
            pl.BlockSpec((1, 2 * DIM), lambda i, j: (0, 0)),
        ],
        out_specs=[
            pl.BlockSpec((1, W2, DIM), lambda i, j: (i * N_WIN + j, 0, 0)),
            pl.BlockSpec((1, W2, 2 * DIM), lambda i, j: (i * N_WIN + j, 0, 0)),
            pl.BlockSpec((WS, WS, DIM), lambda i, j: (i, j, 0)),
            pl.BlockSpec((1, 1, DIM), lambda i, j: (i * N_WIN + j, 0, 0)),
            pl.BlockSpec((1, 1, DIM), lambda i, j: (i * N_WIN + j, 0, 0)),
        ],
        out_shape=[
            jax.ShapeDtypeStruct((P2, W2, DIM), f32),
            jax.ShapeDtypeStruct((P2, W2, 2 * DIM), f32),
            jax.ShapeDtypeStruct((IMG, IMG, DIM), f32),
            jax.ShapeDtypeStruct((P2, 1, DIM), f32),
            jax.ShapeDtypeStruct((P2, 1, DIM), f32),
        ],
    )(x2, wq, wkv, bq, bkv)

    o0, o1, o2, o3 = pl.pallas_call(
        _router_kernel,
        out_shape=[jax.ShapeDtypeStruct((P2, 1), jnp.int32)] * TOPK,
    )(qwin, kwin)
    topk_idx = jnp.concatenate([o0, o1, o2, o3], axis=1)

    def _kv_spec(t):
        return pl.BlockSpec((1, W2, 2 * DIM),
                            lambda p, idx_ref, t=t: (idx_ref[p, t], 0, 0))

    attn_img = pl.pallas_call(
        _attn_kernel,
        grid_spec=pltpu.PrefetchScalarGridSpec(
            num_scalar_prefetch=1,
            grid=(P2,),
            in_specs=[
                pl.BlockSpec((1, W2, DIM), lambda p, idx_ref: (p, 0, 0)),
                _kv_spec(0), _kv_spec(1), _kv_spec(2), _kv_spec(3),
            ],
            out_specs=pl.BlockSpec(
                (WS, WS, DIM), lambda p, idx_ref: (p // N_WIN, p % N_WIN, 0)),
        ),
        out_shape=jax.ShapeDtypeStruct((IMG, IMG, DIM), f32),
    )(topk_idx, q, kv, kv, kv, kv)

    vpad = jnp.pad(vimg, ((2, 2), (2, 2), (0, 0)))
    out = pl.pallas_call(
        _out_kernel,
        grid=(IMG // ROWS,),
        in_specs=[
            pl.BlockSpec((ROWS, IMG, DIM), lambda i: (i, 0, 0)),
            pl.BlockSpec((IMG + 4, IMG + 4, DIM), lambda i: (0, 0, 0)),
            pl.BlockSpec((25, DIM), lambda i: (0, 0)),
            pl.BlockSpec((1, DIM), lambda i: (0, 0)),
            pl.BlockSpec((DIM, DIM), lambda i: (0, 0)),
            pl.BlockSpec((1, DIM), lambda i: (0, 0)),
        ],
        out_specs=pl.BlockSpec((ROWS, IMG, DIM), lambda i: (i, 0, 0)),
        out_shape=jax.ShapeDtypeStruct((IMG, IMG, DIM), f32),
    )(attn_img, vpad, lepe_w.reshape(25, DIM), lepe_b.reshape(1, DIM),
      wo_w, wo_b.reshape(1, DIM))

    return out[None]
